# Initial kernel scaffold; baseline (speedup 1.0000x reference)
import jax
import jax.numpy as jnp
from jax.experimental import pallas as pl


def kernel(xyz, x, conv_w, conv_b, gamma, beta):
    raise NotImplementedError("write your pallas kernel here")


# speedup vs baseline: 13.1721x; 13.1721x over previous
import functools

import jax
import jax.numpy as jnp
from jax.experimental import pallas as pl

_K = 16
_EPS = 1e-5
_RT = 256


def _knn_block(xyzr_ref, xyzc_ref, xt_ref, xsl_ref, w_ref, bc_ref,
               h_ref, s1_ref, s2_ref):
    b = pl.program_id(0)
    t = pl.program_id(1)
    rt = h_ref.shape[2]
    n = xyzc_ref.shape[2]

    xr = xyzr_ref[0]
    xc = xyzc_ref[0]
    sqc = jnp.sum(xc * xc, axis=0, keepdims=True)
    g = jax.lax.dot_general(xr, xc, (((0,), (0,)), ((), ())),
                            preferred_element_type=jnp.float32)
    d = sqc - 2.0 * g

    iota = jax.lax.broadcasted_iota(jnp.int32, (rt, n), 1)
    acc = jnp.zeros((rt, n), dtype=jnp.float32)
    big_i = jnp.int32(n + 1)
    inf = jnp.float32(jnp.inf)
    for _ in range(_K):
        m = jnp.min(d, axis=1, keepdims=True)
        cand = jnp.where(d <= m, iota, big_i)
        am = jnp.min(cand, axis=1, keepdims=True)
        hit = iota == am
        acc = acc + hit.astype(jnp.float32)
        d = jnp.where(hit, inf, d)

    summed_t = jax.lax.dot_general(xt_ref[0], acc, (((0,), (1,)), ((), ())),
                                   preferred_element_type=jnp.float32)
    dx = summed_t - xsl_ref[0]
    h = jax.lax.dot_general(w_ref[...], dx, (((1,), (0,)), ((), ())),
                            preferred_element_type=jnp.float32)
    h = jnp.maximum(h + bc_ref[...], 0.0)
    h_ref[0] = h

    s1 = jnp.sum(h, axis=1, keepdims=True)
    s2 = jnp.sum(h * h, axis=1, keepdims=True)

    @pl.when(jnp.logical_and(b == 0, t == 0))
    def _():
        s1_ref[...] = s1
        s2_ref[...] = s2

    @pl.when(jnp.logical_or(b != 0, t != 0))
    def _():
        s1_ref[...] = s1_ref[...] + s1
        s2_ref[...] = s2_ref[...] + s2


def _bn_block(x_ref, h_ref, s1_ref, s2_ref, g_ref, be_ref, o_ref, *, count):
    inv = jnp.float32(1.0 / count)
    mean = s1_ref[...] * inv
    var = s2_ref[...] * inv - mean * mean
    rstd = jax.lax.rsqrt(var + _EPS)
    scale = g_ref[...] * rstd
    shift = be_ref[...] - mean * scale
    o_ref[0] = x_ref[0] + h_ref[0] * scale + shift


def kernel(xyz, x, conv_w, conv_b, gamma, beta):
    B, C, N = x.shape
    NT = N // _RT

    xyzp = jnp.pad(jnp.transpose(xyz, (0, 2, 1)), ((0, 0), (0, 5), (0, 0)))
    xt = jnp.transpose(x, (0, 2, 1))
    bc = conv_b.reshape(C, 1)
    gc = gamma.reshape(C, 1)
    bec = beta.reshape(C, 1)

    h, s1, s2 = pl.pallas_call(
        _knn_block,
        grid=(B, NT),
        in_specs=[
            pl.BlockSpec((1, 8, _RT), lambda b, t: (b, 0, t)),
            pl.BlockSpec((1, 8, N), lambda b, t: (b, 0, 0)),
            pl.BlockSpec((1, N, C), lambda b, t: (b, 0, 0)),
            pl.BlockSpec((1, C, _RT), lambda b, t: (b, 0, t)),
            pl.BlockSpec((C, C), lambda b, t: (0, 0)),
            pl.BlockSpec((C, 1), lambda b, t: (0, 0)),
        ],
        out_specs=[
            pl.BlockSpec((1, C, _RT), lambda b, t: (b, 0, t)),
            pl.BlockSpec((C, 1), lambda b, t: (0, 0)),
            pl.BlockSpec((C, 1), lambda b, t: (0, 0)),
        ],
        out_shape=[
            jax.ShapeDtypeStruct((B, C, N), jnp.float32),
            jax.ShapeDtypeStruct((C, 1), jnp.float32),
            jax.ShapeDtypeStruct((C, 1), jnp.float32),
        ],
    )(xyzp, xyzp, xt, x, conv_w, bc)

    out = pl.pallas_call(
        functools.partial(_bn_block, count=B * N),
        grid=(B,),
        in_specs=[
            pl.BlockSpec((1, C, N), lambda b: (b, 0, 0)),
            pl.BlockSpec((1, C, N), lambda b: (b, 0, 0)),
            pl.BlockSpec((C, 1), lambda b: (0, 0)),
            pl.BlockSpec((C, 1), lambda b: (0, 0)),
            pl.BlockSpec((C, 1), lambda b: (0, 0)),
            pl.BlockSpec((C, 1), lambda b: (0, 0)),
        ],
        out_specs=pl.BlockSpec((1, C, N), lambda b: (b, 0, 0)),
        out_shape=jax.ShapeDtypeStruct((B, C, N), jnp.float32),
    )(x, h, s1, s2, gc, bec)
    return out
